# baseline (speedup 1.0000x reference)
import functools

import jax
import jax.numpy as jnp
import numpy as np
from jax import lax
from jax.experimental import pallas as pl
from jax.experimental.pallas import tpu as pltpu
from jax.experimental.pallas import tpu_sc as plsc

N = 10000
E = 320000
D_IN = 128
D_HID = 16
N_CLASSES = 40

NC, NS, L = 2, 16, 16
NW = NC * NS
CH = 128
NCH = 80
EPT = NCH * CH
EPAD = NW * EPT

NP = 10240
RPT = NP // NS

CPACK = 48
_MESH = plsc.VectorSubcoreMesh(core_axis_name="c", subcore_axis_name="s")
_SC_PARAMS = pltpu.CompilerParams(use_tc_tiling_on_sc=False,
                                  needs_layout_passes=False)


def _fast_rsqrt(d):
    i = plsc.bitcast(d, jnp.int32)
    y = plsc.bitcast(jnp.int32(0x5F3759DF) - (i >> 1), jnp.float32)
    for _ in range(3):
        y = y * (1.5 - 0.5 * d * y * y)
    return y


def _fill_zero(ref):
    @pl.loop(0, RPT, unroll=8)
    def _(i):
        ref[i, :] = jnp.zeros((L,), jnp.float32)


@functools.partial(
    pl.kernel,
    out_type=jax.ShapeDtypeStruct((NC, NP, L), jnp.float32),
    mesh=_MESH,
    scratch_types=[
        pltpu.VMEM((NCH, CH), jnp.int32),
        pltpu.VMEM((CH, L), jnp.float32),
        pltpu.VMEM((RPT, L), jnp.float32),
        pltpu.VMEM_SHARED((NP, L), jnp.float32),
        pltpu.SemaphoreType.DMA,
        pltpu.SemaphoreType.DMA,
        pltpu.SemaphoreType.DMA,
        pltpu.SemaphoreType.DMA,
    ],
    compiler_params=_SC_PARAMS,
)
def _deg_kernel(edges_hbm, out_hbm, dst_v, ones_v, zero_v, acc_sh,
                ss0, ss1, ss2, ss3):
    cid = lax.axis_index("c")
    sid = lax.axis_index("s")
    wid = cid * NS + sid
    ssems = (ss0, ss1, ss2, ss3)
    pltpu.sync_copy(edges_hbm.at[1, wid], dst_v)

    @pl.loop(0, CH, unroll=8)
    def _(i):
        ones_v[i, :] = jnp.ones((L,), jnp.float32)
    _fill_zero(zero_v)
    pltpu.sync_copy(zero_v, acc_sh.at[pl.ds(sid * RPT, RPT)])
    plsc.subcore_barrier()

    @pl.loop(0, NCH, step=4)
    def _(j):
        for k in range(4):
            idx = j + k

            @pl.when(idx >= 4)
            def _():
                pltpu.make_async_copy(ones_v, acc_sh.at[dst_v.at[0]],
                                      ssems[k]).wait()
            pltpu.async_copy(ones_v, acc_sh.at[dst_v.at[idx]], ssems[k],
                             add=True)
    for k in range(4):
        pltpu.make_async_copy(ones_v, acc_sh.at[dst_v.at[0]], ssems[k]).wait()
    plsc.subcore_barrier()
    pltpu.sync_copy(acc_sh.at[pl.ds(sid * RPT, RPT)],
                    out_hbm.at[cid, pl.ds(sid * RPT, RPT)])


def _agg_pipeline(src_v, dst_v, buf, q_sh, s_sh, gsems, ssems):
    def gather(idx, k):
        pltpu.async_copy(q_sh.at[src_v.at[idx]], buf.at[k], gsems[k])

    def wait_gather(k):
        pltpu.make_async_copy(q_sh.at[src_v.at[0]], buf.at[k],
                              gsems[k]).wait()

    def scatter(idx, k):
        pltpu.async_copy(buf.at[k], s_sh.at[dst_v.at[idx]], ssems[k],
                         add=True)

    def wait_scatter(k):
        pltpu.make_async_copy(buf.at[k], s_sh.at[dst_v.at[0]],
                              ssems[k]).wait()

    gather(0, 0)
    gather(1, 1)

    @pl.loop(0, NCH, step=4)
    def _(j):
        for k in range(4):
            idx = j + k
            wait_gather(k)
            scatter(idx, k)
            k2 = (k + 2) % 4

            @pl.when(idx + 2 < NCH)
            def _():
                @pl.when(idx >= 2)
                def _():
                    wait_scatter(k2)
                gather(idx + 2, k2)
    for k in range(4):
        wait_scatter(k)


_AGG_SCRATCH = [
    pltpu.VMEM((NCH, CH), jnp.int32),
    pltpu.VMEM((NCH, CH), jnp.int32),
    pltpu.VMEM((4, CH, L), jnp.float32),
    pltpu.VMEM((RPT, L), jnp.float32),
    pltpu.VMEM_SHARED((NP, L), jnp.float32),
    pltpu.VMEM_SHARED((NP, L), jnp.float32),
    pltpu.SemaphoreType.DMA,
    pltpu.SemaphoreType.DMA,
    pltpu.SemaphoreType.DMA,
    pltpu.SemaphoreType.DMA,
    pltpu.SemaphoreType.DMA,
    pltpu.SemaphoreType.DMA,
    pltpu.SemaphoreType.DMA,
    pltpu.SemaphoreType.DMA,
]


@functools.partial(
    pl.kernel,
    out_type=(jax.ShapeDtypeStruct((NC, NP, L), jnp.float32),
              jax.ShapeDtypeStruct((NP, L), jnp.float32),
              jax.ShapeDtypeStruct((NP, L), jnp.float32)),
    mesh=_MESH,
    scratch_types=_AGG_SCRATCH + [
        pltpu.VMEM((RPT, L), jnp.float32),
        pltpu.VMEM((RPT, L), jnp.float32),
        pltpu.VMEM((RPT, L), jnp.float32),
        pltpu.VMEM((RPT, L), jnp.float32),
    ],
    compiler_params=_SC_PARAMS,
)
def _layer1_kernel(degp_hbm, p_hbm, edges_hbm,
                   out_hbm, q1_hbm, dinv_hbm,
                   src_v, dst_v, buf, zero_v, q_sh, s_sh,
                   gs0, gs1, gs2, gs3, ss0, ss1, ss2, ss3,
                   p_v, d0_v, d1_v, q_v):
    cid = lax.axis_index("c")
    sid = lax.axis_index("s")
    wid = cid * NS + sid
    rows = pl.ds(sid * RPT, RPT)
    pltpu.sync_copy(edges_hbm.at[0, wid], src_v)
    pltpu.sync_copy(edges_hbm.at[1, wid], dst_v)
    pltpu.sync_copy(p_hbm.at[rows], p_v)
    pltpu.sync_copy(degp_hbm.at[0, rows], d0_v)
    pltpu.sync_copy(degp_hbm.at[1, rows], d1_v)

    @pl.loop(0, RPT, unroll=8)
    def _(i):
        d = d0_v[i, :] + d1_v[i, :] + 1.0
        y = _fast_rsqrt(d)
        d1_v[i, :] = y
        q_v[i, :] = y * p_v[i, :]

    pltpu.sync_copy(q_v, q_sh.at[rows])

    @pl.when(cid == 0)
    def _():
        pltpu.sync_copy(q_v, q1_hbm.at[rows])
        pltpu.sync_copy(d1_v, dinv_hbm.at[rows])

    _fill_zero(zero_v)
    pltpu.sync_copy(zero_v, s_sh.at[rows])
    plsc.subcore_barrier()
    _agg_pipeline(src_v, dst_v, buf, q_sh, s_sh,
                  (gs0, gs1, gs2, gs3), (ss0, ss1, ss2, ss3))
    plsc.subcore_barrier()
    pltpu.sync_copy(s_sh.at[rows], out_hbm.at[cid, rows])


@functools.partial(
    pl.kernel,
    out_type=jax.ShapeDtypeStruct((NC, NP, L), jnp.float32),
    mesh=_MESH,
    scratch_types=_AGG_SCRATCH + [
        pltpu.VMEM((RPT, L), jnp.float32),
        pltpu.VMEM((RPT, L), jnp.float32),
        pltpu.VMEM((RPT, L), jnp.float32),
        pltpu.VMEM((RPT, L), jnp.float32),
        pltpu.VMEM((RPT, L), jnp.float32),
        pltpu.VMEM((L,), jnp.float32),
    ],
    compiler_params=_SC_PARAMS,
)
def _layer2_kernel(s1p_hbm, q1_hbm, dinv_hbm, b1_hbm, edges_hbm,
                   out_hbm,
                   src_v, dst_v, buf, zero_v, q_sh, s_sh,
                   gs0, gs1, gs2, gs3, ss0, ss1, ss2, ss3,
                   s0_v, s1_v, q1_v, dinv_v, q2_v, b1_v):
    cid = lax.axis_index("c")
    sid = lax.axis_index("s")
    wid = cid * NS + sid
    rows = pl.ds(sid * RPT, RPT)
    pltpu.sync_copy(edges_hbm.at[0, wid], src_v)
    pltpu.sync_copy(edges_hbm.at[1, wid], dst_v)
    pltpu.sync_copy(s1p_hbm.at[0, rows], s0_v)
    pltpu.sync_copy(s1p_hbm.at[1, rows], s1_v)
    pltpu.sync_copy(q1_hbm.at[rows], q1_v)
    pltpu.sync_copy(dinv_hbm.at[rows], dinv_v)
    pltpu.sync_copy(b1_hbm, b1_v)

    @pl.loop(0, RPT, unroll=8)
    def _(i):
        y = dinv_v[i, :]
        agg = y * (s0_v[i, :] + s1_v[i, :] + q1_v[i, :])
        h = jnp.maximum(agg + b1_v[...], 0.0)
        q2_v[i, :] = y * h

    pltpu.sync_copy(q2_v, q_sh.at[rows])
    _fill_zero(zero_v)
    pltpu.sync_copy(zero_v, s_sh.at[rows])
    plsc.subcore_barrier()
    _agg_pipeline(src_v, dst_v, buf, q_sh, s_sh,
                  (gs0, gs1, gs2, gs3), (ss0, ss1, ss2, ss3))
    plsc.subcore_barrier()

    pltpu.sync_copy(s_sh.at[rows], s0_v)

    @pl.when(cid == 0)
    def _():
        @pl.loop(0, RPT, unroll=8)
        def _(i):
            s0_v[i, :] = dinv_v[i, :] * (s0_v[i, :] + q2_v[i, :])

    @pl.when(cid != 0)
    def _():
        @pl.loop(0, RPT, unroll=8)
        def _(i):
            s0_v[i, :] = dinv_v[i, :] * s0_v[i, :]
    pltpu.sync_copy(s0_v, out_hbm.at[cid, rows])


def _edge_body(e_ref, o_ref):
    o_ref[:, :2500, :] = e_ref[...].reshape(2, 2500, 128)
    o_ref[:, 2500:, :] = jnp.full((2, 60, 128), NP - 1, jnp.int32)


_tc_edges = pl.pallas_call(
    _edge_body,
    in_specs=[pl.BlockSpec((2, E), lambda: (0, 0))],
    out_specs=pl.BlockSpec((2, 2560, 128), lambda: (0, 0, 0)),
    out_shape=jax.ShapeDtypeStruct((2, 2560, 128), jnp.int32),
)


def _mm1_body(x_ref, w_ref, o_ref):
    o_ref[...] = jnp.dot(x_ref[...], w_ref[...],
                         preferred_element_type=jnp.float32)


_tc_mm1 = pl.pallas_call(
    _mm1_body,
    grid=(5,),
    in_specs=[pl.BlockSpec((2048, D_IN), lambda i: (i, 0)),
              pl.BlockSpec((D_IN, D_HID), lambda i: (0, 0))],
    out_specs=pl.BlockSpec((2048, D_HID), lambda i: (i, 0)),
    out_shape=jax.ShapeDtypeStruct((NP, D_HID), jnp.float32),
)


def _mm2_body(p_ref, w_ref, b_ref, g_ref, gt_ref, o_ref):
    p = p_ref[0] + p_ref[1]
    z = jnp.dot(p, w_ref[...], preferred_element_type=jnp.float32)
    z = z + b_ref[...]
    m = jnp.max(z, axis=1, keepdims=True)
    e = jnp.exp(z - m)
    s = jnp.dot(e, g_ref[...], preferred_element_type=jnp.float32)
    logs = jnp.log(s)
    o_ref[...] = (z - m) - jnp.dot(logs, gt_ref[...],
                                   preferred_element_type=jnp.float32)


_PBLK = 256
_tc_mm2 = pl.pallas_call(
    _mm2_body,
    grid=(NP // 8 // _PBLK,),
    in_specs=[pl.BlockSpec((NC, _PBLK, 128), lambda i: (0, i, 0)),
              pl.BlockSpec((128, 8 * CPACK), lambda i: (0, 0)),
              pl.BlockSpec((1, 8 * CPACK), lambda i: (0, 0)),
              pl.BlockSpec((8 * CPACK, 8), lambda i: (0, 0)),
              pl.BlockSpec((8, 8 * CPACK), lambda i: (0, 0))],
    out_specs=pl.BlockSpec((_PBLK, 8 * CPACK), lambda i: (i, 0)),
    out_shape=jax.ShapeDtypeStruct((NP // 8, 8 * CPACK), jnp.float32),
)

_G_NP = np.kron(np.eye(8, dtype=np.float32),
                np.pad(np.ones((N_CLASSES, 1), np.float32),
                       ((0, CPACK - N_CLASSES), (0, 0))))
_GT_NP = _G_NP.T.copy()


def kernel(x, edge_index, W1, b1, W2, b2):
    edges = _tc_edges(edge_index).reshape(2, NW, NCH, CH)

    degp = _deg_kernel(edges)
    p = _tc_mm1(x, W1)
    s1p, q1, dinv = _layer1_kernel(degp, p, edges)
    s2p = _layer2_kernel(s1p, q1, dinv, b1, edges)

    w2big = jnp.kron(jnp.eye(8, dtype=jnp.float32),
                     jnp.pad(W2, ((0, 0), (0, CPACK - N_CLASSES))))
    b2big = jnp.tile(jnp.pad(b2, (0, CPACK - N_CLASSES)), 8)[None, :]
    out_pack = _tc_mm2(s2p.reshape(NC, NP // 8, 128), w2big, b2big,
                       jnp.asarray(_G_NP), jnp.asarray(_GT_NP))
    return out_pack.reshape(NP, CPACK)[:N, :N_CLASSES]


# speedup vs baseline: 1.1341x; 1.1341x over previous
import functools

import jax
import jax.numpy as jnp
import numpy as np
from jax import lax
from jax.experimental import pallas as pl
from jax.experimental.pallas import tpu as pltpu
from jax.experimental.pallas import tpu_sc as plsc

N = 10000
E = 320000
D_IN = 128
D_HID = 16
N_CLASSES = 40

NC, NS, L = 2, 16, 16
NW = NC * NS
CH = 128
NCH = 80
EPT = NCH * CH
EPAD = NW * EPT

NP = 10240
RPT = NP // NS

CPACK = 48
_MESH = plsc.VectorSubcoreMesh(core_axis_name="c", subcore_axis_name="s")
_SC_PARAMS = pltpu.CompilerParams(use_tc_tiling_on_sc=False,
                                  needs_layout_passes=False)


def _fast_rsqrt(d):
    i = plsc.bitcast(d, jnp.int32)
    y = plsc.bitcast(jnp.int32(0x5F3759DF) - (i >> 1), jnp.float32)
    for _ in range(3):
        y = y * (1.5 - 0.5 * d * y * y)
    return y


def _fill_zero(ref):
    @pl.loop(0, RPT, unroll=8)
    def _(i):
        ref[i, :] = jnp.zeros((L,), jnp.float32)


@functools.partial(
    pl.kernel,
    out_type=jax.ShapeDtypeStruct((NC, NP, L), jnp.float32),
    mesh=_MESH,
    scratch_types=[
        pltpu.VMEM((NCH, CH), jnp.int32),
        pltpu.VMEM((CH, L), jnp.float32),
        pltpu.VMEM((RPT, L), jnp.float32),
        pltpu.VMEM_SHARED((NP, L), jnp.float32),
        pltpu.SemaphoreType.DMA,
        pltpu.SemaphoreType.DMA,
        pltpu.SemaphoreType.DMA,
        pltpu.SemaphoreType.DMA,
    ],
    compiler_params=_SC_PARAMS,
)
def _deg_kernel(edges_hbm, out_hbm, dst_v, ones_v, zero_v, acc_sh,
                ss0, ss1, ss2, ss3):
    cid = lax.axis_index("c")
    sid = lax.axis_index("s")
    wid = cid * NS + sid
    ssems = (ss0, ss1, ss2, ss3)
    pltpu.sync_copy(edges_hbm.at[1, wid], dst_v)

    @pl.loop(0, CH, unroll=8)
    def _(i):
        ones_v[i, :] = jnp.ones((L,), jnp.float32)
    _fill_zero(zero_v)
    pltpu.sync_copy(zero_v, acc_sh.at[pl.ds(sid * RPT, RPT)])
    plsc.subcore_barrier()

    @pl.loop(0, NCH, step=4)
    def _(j):
        for k in range(4):
            idx = j + k

            @pl.when(idx >= 4)
            def _():
                pltpu.make_async_copy(ones_v, acc_sh.at[dst_v.at[0]],
                                      ssems[k]).wait()
            pltpu.async_copy(ones_v, acc_sh.at[dst_v.at[idx]], ssems[k],
                             add=True)
    for k in range(4):
        pltpu.make_async_copy(ones_v, acc_sh.at[dst_v.at[0]], ssems[k]).wait()
    plsc.subcore_barrier()
    pltpu.sync_copy(acc_sh.at[pl.ds(sid * RPT, RPT)],
                    out_hbm.at[cid, pl.ds(sid * RPT, RPT)])


def _agg_pipeline(src_v, dst_v, buf, q_sh, s_sh, gsems, ssems):
    def gather(idx, k):
        pltpu.async_copy(q_sh.at[src_v.at[idx]], buf.at[k], gsems[k])

    def wait_gather(k):
        pltpu.make_async_copy(q_sh.at[src_v.at[0]], buf.at[k],
                              gsems[k]).wait()

    def scatter(idx, k):
        pltpu.async_copy(buf.at[k], s_sh.at[dst_v.at[idx]], ssems[k],
                         add=True)

    def wait_scatter(k):
        pltpu.make_async_copy(buf.at[k], s_sh.at[dst_v.at[0]],
                              ssems[k]).wait()

    gather(0, 0)
    gather(1, 1)

    @pl.loop(0, NCH, step=4)
    def _(j):
        for k in range(4):
            idx = j + k
            wait_gather(k)
            scatter(idx, k)
            k2 = (k + 2) % 4

            @pl.when(idx + 2 < NCH)
            def _():
                @pl.when(idx >= 2)
                def _():
                    wait_scatter(k2)
                gather(idx + 2, k2)
    for k in range(4):
        wait_scatter(k)


_AGG_SCRATCH = [
    pltpu.VMEM((NCH, CH), jnp.int32),
    pltpu.VMEM((NCH, CH), jnp.int32),
    pltpu.VMEM((4, CH, L), jnp.float32),
    pltpu.VMEM((RPT, L), jnp.float32),
    pltpu.VMEM_SHARED((NP, L), jnp.float32),
    pltpu.VMEM_SHARED((NP, L), jnp.float32),
    pltpu.SemaphoreType.DMA,
    pltpu.SemaphoreType.DMA,
    pltpu.SemaphoreType.DMA,
    pltpu.SemaphoreType.DMA,
    pltpu.SemaphoreType.DMA,
    pltpu.SemaphoreType.DMA,
    pltpu.SemaphoreType.DMA,
    pltpu.SemaphoreType.DMA,
]


@functools.partial(
    pl.kernel,
    out_type=(jax.ShapeDtypeStruct((NC, NP, L), jnp.float32),
              jax.ShapeDtypeStruct((NP, L), jnp.float32),
              jax.ShapeDtypeStruct((NP, L), jnp.float32)),
    mesh=_MESH,
    scratch_types=_AGG_SCRATCH + [
        pltpu.VMEM((RPT, L), jnp.float32),
        pltpu.VMEM((RPT, L), jnp.float32),
        pltpu.VMEM((RPT, L), jnp.float32),
        pltpu.VMEM((RPT, L), jnp.float32),
    ],
    compiler_params=_SC_PARAMS,
)
def _layer1_kernel(degp_hbm, p_hbm, edges_hbm,
                   out_hbm, q1_hbm, dinv_hbm,
                   src_v, dst_v, buf, zero_v, q_sh, s_sh,
                   gs0, gs1, gs2, gs3, ss0, ss1, ss2, ss3,
                   p_v, d0_v, d1_v, q_v):
    cid = lax.axis_index("c")
    sid = lax.axis_index("s")
    wid = cid * NS + sid
    rows = pl.ds(sid * RPT, RPT)
    pltpu.sync_copy(edges_hbm.at[0, wid], src_v)
    pltpu.sync_copy(edges_hbm.at[1, wid], dst_v)
    pltpu.sync_copy(p_hbm.at[rows], p_v)
    pltpu.sync_copy(degp_hbm.at[0, rows], d0_v)
    pltpu.sync_copy(degp_hbm.at[1, rows], d1_v)

    @pl.loop(0, RPT, unroll=8)
    def _(i):
        d = d0_v[i, :] + d1_v[i, :] + 1.0
        y = _fast_rsqrt(d)
        d1_v[i, :] = y
        q_v[i, :] = y * p_v[i, :]

    pltpu.sync_copy(q_v, q_sh.at[rows])

    @pl.when(cid == 0)
    def _():
        pltpu.sync_copy(q_v, q1_hbm.at[rows])
        pltpu.sync_copy(d1_v, dinv_hbm.at[rows])

    _fill_zero(zero_v)
    pltpu.sync_copy(zero_v, s_sh.at[rows])
    plsc.subcore_barrier()
    _agg_pipeline(src_v, dst_v, buf, q_sh, s_sh,
                  (gs0, gs1, gs2, gs3), (ss0, ss1, ss2, ss3))
    plsc.subcore_barrier()
    pltpu.sync_copy(s_sh.at[rows], out_hbm.at[cid, rows])


@functools.partial(
    pl.kernel,
    out_type=jax.ShapeDtypeStruct((NC, NP, L), jnp.float32),
    mesh=_MESH,
    scratch_types=_AGG_SCRATCH + [
        pltpu.VMEM((RPT, L), jnp.float32),
        pltpu.VMEM((RPT, L), jnp.float32),
        pltpu.VMEM((RPT, L), jnp.float32),
        pltpu.VMEM((RPT, L), jnp.float32),
        pltpu.VMEM((RPT, L), jnp.float32),
        pltpu.VMEM((L,), jnp.float32),
    ],
    compiler_params=_SC_PARAMS,
)
def _layer2_kernel(s1p_hbm, q1_hbm, dinv_hbm, b1_hbm, edges_hbm,
                   out_hbm,
                   src_v, dst_v, buf, zero_v, q_sh, s_sh,
                   gs0, gs1, gs2, gs3, ss0, ss1, ss2, ss3,
                   s0_v, s1_v, q1_v, dinv_v, q2_v, b1_v):
    cid = lax.axis_index("c")
    sid = lax.axis_index("s")
    wid = cid * NS + sid
    rows = pl.ds(sid * RPT, RPT)
    pltpu.sync_copy(edges_hbm.at[0, wid], src_v)
    pltpu.sync_copy(edges_hbm.at[1, wid], dst_v)
    pltpu.sync_copy(s1p_hbm.at[0, rows], s0_v)
    pltpu.sync_copy(s1p_hbm.at[1, rows], s1_v)
    pltpu.sync_copy(q1_hbm.at[rows], q1_v)
    pltpu.sync_copy(dinv_hbm.at[rows], dinv_v)
    pltpu.sync_copy(b1_hbm, b1_v)

    @pl.loop(0, RPT, unroll=8)
    def _(i):
        y = dinv_v[i, :]
        agg = y * (s0_v[i, :] + s1_v[i, :] + q1_v[i, :])
        h = jnp.maximum(agg + b1_v[...], 0.0)
        q2_v[i, :] = y * h

    pltpu.sync_copy(q2_v, q_sh.at[rows])
    _fill_zero(zero_v)
    pltpu.sync_copy(zero_v, s_sh.at[rows])
    plsc.subcore_barrier()
    _agg_pipeline(src_v, dst_v, buf, q_sh, s_sh,
                  (gs0, gs1, gs2, gs3), (ss0, ss1, ss2, ss3))
    plsc.subcore_barrier()

    pltpu.sync_copy(s_sh.at[rows], s0_v)

    @pl.when(cid == 0)
    def _():
        @pl.loop(0, RPT, unroll=8)
        def _(i):
            s0_v[i, :] = dinv_v[i, :] * (s0_v[i, :] + q2_v[i, :])

    @pl.when(cid != 0)
    def _():
        @pl.loop(0, RPT, unroll=8)
        def _(i):
            s0_v[i, :] = dinv_v[i, :] * s0_v[i, :]
    pltpu.sync_copy(s0_v, out_hbm.at[cid, rows])


def _edge_body(e_ref, o_ref):
    o_ref[:, :2500, :] = e_ref[...].reshape(2, 2500, 128)
    lane = jax.lax.broadcasted_iota(jnp.int32, (2, 60, 128), 2)
    row = jax.lax.broadcasted_iota(jnp.int32, (2, 60, 128), 1)
    o_ref[:, 2500:, :] = N + (row * 128 + lane) % (NP - N)


_tc_edges = pl.pallas_call(
    _edge_body,
    in_specs=[pl.BlockSpec((2, E), lambda: (0, 0))],
    out_specs=pl.BlockSpec((2, 2560, 128), lambda: (0, 0, 0)),
    out_shape=jax.ShapeDtypeStruct((2, 2560, 128), jnp.int32),
)


def _mm1_body(x_ref, w_ref, o_ref):
    o_ref[...] = jnp.dot(x_ref[...], w_ref[...],
                         preferred_element_type=jnp.float32)


_tc_mm1 = pl.pallas_call(
    _mm1_body,
    grid=(5,),
    in_specs=[pl.BlockSpec((2048, D_IN), lambda i: (i, 0)),
              pl.BlockSpec((D_IN, D_HID), lambda i: (0, 0))],
    out_specs=pl.BlockSpec((2048, D_HID), lambda i: (i, 0)),
    out_shape=jax.ShapeDtypeStruct((NP, D_HID), jnp.float32),
)


def _mm2_body(p_ref, w_ref, b_ref, g_ref, gt_ref, o_ref):
    p = p_ref[0] + p_ref[1]
    z = jnp.dot(p, w_ref[...], preferred_element_type=jnp.float32)
    z = z + b_ref[...]
    m = jnp.max(z, axis=1, keepdims=True)
    e = jnp.exp(z - m)
    s = jnp.dot(e, g_ref[...], preferred_element_type=jnp.float32)
    logs = jnp.log(s)
    o_ref[...] = (z - m) - jnp.dot(logs, gt_ref[...],
                                   preferred_element_type=jnp.float32)


_PBLK = 256
_tc_mm2 = pl.pallas_call(
    _mm2_body,
    grid=(NP // 8 // _PBLK,),
    in_specs=[pl.BlockSpec((NC, _PBLK, 128), lambda i: (0, i, 0)),
              pl.BlockSpec((128, 8 * CPACK), lambda i: (0, 0)),
              pl.BlockSpec((1, 8 * CPACK), lambda i: (0, 0)),
              pl.BlockSpec((8 * CPACK, 8), lambda i: (0, 0)),
              pl.BlockSpec((8, 8 * CPACK), lambda i: (0, 0))],
    out_specs=pl.BlockSpec((_PBLK, 8 * CPACK), lambda i: (i, 0)),
    out_shape=jax.ShapeDtypeStruct((NP // 8, 8 * CPACK), jnp.float32),
)

_G_NP = np.kron(np.eye(8, dtype=np.float32),
                np.pad(np.ones((N_CLASSES, 1), np.float32),
                       ((0, CPACK - N_CLASSES), (0, 0))))
_GT_NP = _G_NP.T.copy()


def kernel(x, edge_index, W1, b1, W2, b2):
    edges = _tc_edges(edge_index).reshape(2, NW, NCH, CH)

    degp = _deg_kernel(edges)
    p = _tc_mm1(x, W1)
    s1p, q1, dinv = _layer1_kernel(degp, p, edges)
    s2p = _layer2_kernel(s1p, q1, dinv, b1, edges)

    w2big = jnp.kron(jnp.eye(8, dtype=jnp.float32),
                     jnp.pad(W2, ((0, 0), (0, CPACK - N_CLASSES))))
    b2big = jnp.tile(jnp.pad(b2, (0, CPACK - N_CLASSES)), 8)[None, :]
    out_pack = _tc_mm2(s2p.reshape(NC, NP // 8, 128), w2big, b2big,
                       jnp.asarray(_G_NP), jnp.asarray(_GT_NP))
    return out_pack.reshape(NP, CPACK)[:N, :N_CLASSES]


# speedup vs baseline: 1.1964x; 1.0550x over previous
import functools

import jax
import jax.numpy as jnp
import numpy as np
from jax import lax
from jax.experimental import pallas as pl
from jax.experimental.pallas import tpu as pltpu
from jax.experimental.pallas import tpu_sc as plsc

N = 10000
E = 320000
D_IN = 128
D_HID = 16
N_CLASSES = 40

NC, NS, L = 2, 16, 16
NW = NC * NS
CH = 128
NCH = 80
EPT = NCH * CH
EPAD = NW * EPT

NP = 10240
RPT = NP // NS

CPACK = 48
_MESH = plsc.VectorSubcoreMesh(core_axis_name="c", subcore_axis_name="s")
_SC_PARAMS = pltpu.CompilerParams(use_tc_tiling_on_sc=False,
                                  needs_layout_passes=False)


def _fast_rsqrt(d):
    i = plsc.bitcast(d, jnp.int32)
    y = plsc.bitcast(jnp.int32(0x5F3759DF) - (i >> 1), jnp.float32)
    for _ in range(3):
        y = y * (1.5 - 0.5 * d * y * y)
    return y


def _fill_zero(ref):
    @pl.loop(0, RPT, unroll=8)
    def _(i):
        ref[i, :] = jnp.zeros((L,), jnp.float32)


@functools.partial(
    pl.kernel,
    out_type=jax.ShapeDtypeStruct((NC, NP, L), jnp.float32),
    mesh=_MESH,
    scratch_types=[
        pltpu.VMEM((NCH, CH), jnp.int32),
        pltpu.VMEM((CH, L), jnp.float32),
        pltpu.VMEM((RPT, L), jnp.float32),
        pltpu.VMEM_SHARED((NP, L), jnp.float32),
        pltpu.SemaphoreType.DMA,
        pltpu.SemaphoreType.DMA,
        pltpu.SemaphoreType.DMA,
        pltpu.SemaphoreType.DMA,
    ],
    compiler_params=_SC_PARAMS,
)
def _deg_kernel(edges_hbm, out_hbm, dst_v, ones_v, zero_v, acc_sh,
                ss0, ss1, ss2, ss3):
    cid = lax.axis_index("c")
    sid = lax.axis_index("s")
    wid = cid * NS + sid
    ssems = (ss0, ss1, ss2, ss3)
    pltpu.sync_copy(edges_hbm.at[1, wid], dst_v)

    @pl.loop(0, CH, unroll=8)
    def _(i):
        ones_v[i, :] = jnp.ones((L,), jnp.float32)
    _fill_zero(zero_v)
    pltpu.sync_copy(zero_v, acc_sh.at[pl.ds(sid * RPT, RPT)])
    plsc.subcore_barrier()

    @pl.loop(0, NCH, step=4)
    def _(j):
        for k in range(4):
            idx = j + k

            @pl.when(idx >= 4)
            def _():
                pltpu.make_async_copy(ones_v, acc_sh.at[dst_v.at[0]],
                                      ssems[k]).wait()
            pltpu.async_copy(ones_v, acc_sh.at[dst_v.at[idx]], ssems[k],
                             add=True)
    for k in range(4):
        pltpu.make_async_copy(ones_v, acc_sh.at[dst_v.at[0]], ssems[k]).wait()
    plsc.subcore_barrier()
    pltpu.sync_copy(acc_sh.at[pl.ds(sid * RPT, RPT)],
                    out_hbm.at[cid, pl.ds(sid * RPT, RPT)])


def _agg_pipeline(src_v, dst_v, buf, q_sh, s_sh, gsems, ssems):
    def gather(idx, k):
        pltpu.async_copy(q_sh.at[src_v.at[idx]], buf.at[k], gsems[k])

    def wait_gather(k):
        pltpu.make_async_copy(q_sh.at[src_v.at[0]], buf.at[k],
                              gsems[k]).wait()

    def scatter(idx, k):
        pltpu.async_copy(buf.at[k], s_sh.at[dst_v.at[idx]], ssems[k],
                         add=True)

    def wait_scatter(k):
        pltpu.make_async_copy(buf.at[k], s_sh.at[dst_v.at[0]],
                              ssems[k]).wait()

    gather(0, 0)
    gather(1, 1)

    @pl.loop(0, NCH, step=4)
    def _(j):
        for k in range(4):
            idx = j + k
            wait_gather(k)
            scatter(idx, k)
            k2 = (k + 2) % 4

            @pl.when(idx + 2 < NCH)
            def _():
                @pl.when(idx >= 2)
                def _():
                    wait_scatter(k2)
                gather(idx + 2, k2)
    for k in range(4):
        wait_scatter(k)


_AGG_SCRATCH = [
    pltpu.VMEM((NCH, CH), jnp.int32),
    pltpu.VMEM((NCH, CH), jnp.int32),
    pltpu.VMEM((4, CH, L), jnp.float32),
    pltpu.VMEM((RPT, L), jnp.float32),
    pltpu.VMEM_SHARED((NP, L), jnp.float32),
    pltpu.VMEM_SHARED((NP, L), jnp.float32),
    pltpu.SemaphoreType.DMA,
    pltpu.SemaphoreType.DMA,
    pltpu.SemaphoreType.DMA,
    pltpu.SemaphoreType.DMA,
    pltpu.SemaphoreType.DMA,
    pltpu.SemaphoreType.DMA,
    pltpu.SemaphoreType.DMA,
    pltpu.SemaphoreType.DMA,
]


@functools.partial(
    pl.kernel,
    out_type=(jax.ShapeDtypeStruct((NC, NP, L), jnp.float32),
              jax.ShapeDtypeStruct((NP, L), jnp.float32),
              jax.ShapeDtypeStruct((NP, L), jnp.float32)),
    mesh=_MESH,
    scratch_types=_AGG_SCRATCH + [
        pltpu.VMEM((RPT, L), jnp.float32),
        pltpu.VMEM((RPT, L), jnp.float32),
        pltpu.VMEM((RPT, L), jnp.float32),
        pltpu.VMEM((RPT, L), jnp.float32),
    ],
    compiler_params=_SC_PARAMS,
)
def _layer1_kernel(degp_hbm, p_hbm, edges_hbm,
                   out_hbm, q1_hbm, dinv_hbm,
                   src_v, dst_v, buf, zero_v, q_sh, s_sh,
                   gs0, gs1, gs2, gs3, ss0, ss1, ss2, ss3,
                   p_v, d0_v, d1_v, q_v):
    cid = lax.axis_index("c")
    sid = lax.axis_index("s")
    wid = cid * NS + sid
    rows = pl.ds(sid * RPT, RPT)
    e0 = pltpu.async_copy(edges_hbm.at[0, wid], src_v, gs0)
    e1 = pltpu.async_copy(edges_hbm.at[1, wid], dst_v, gs1)
    cp = pltpu.async_copy(p_hbm.at[rows], p_v, gs2)
    c0 = pltpu.async_copy(degp_hbm.at[0, rows], d0_v, gs3)
    c1 = pltpu.async_copy(degp_hbm.at[1, rows], d1_v, ss0)
    _fill_zero(zero_v)
    cz = pltpu.async_copy(zero_v, s_sh.at[rows], ss1)
    cp.wait()
    c0.wait()
    c1.wait()

    @pl.loop(0, RPT, unroll=8)
    def _(i):
        d = d0_v[i, :] + d1_v[i, :] + 1.0
        y = _fast_rsqrt(d)
        d1_v[i, :] = y
        q_v[i, :] = y * p_v[i, :]

    pltpu.sync_copy(q_v, q_sh.at[rows])

    @pl.when(cid == 0)
    def _():
        pltpu.sync_copy(q_v, q1_hbm.at[rows])
        pltpu.sync_copy(d1_v, dinv_hbm.at[rows])

    e0.wait()
    e1.wait()
    cz.wait()
    plsc.subcore_barrier()
    _agg_pipeline(src_v, dst_v, buf, q_sh, s_sh,
                  (gs0, gs1, gs2, gs3), (ss0, ss1, ss2, ss3))
    plsc.subcore_barrier()
    pltpu.sync_copy(s_sh.at[rows], out_hbm.at[cid, rows])


@functools.partial(
    pl.kernel,
    out_type=jax.ShapeDtypeStruct((NC, NP, L), jnp.float32),
    mesh=_MESH,
    scratch_types=_AGG_SCRATCH + [
        pltpu.VMEM((RPT, L), jnp.float32),
        pltpu.VMEM((RPT, L), jnp.float32),
        pltpu.VMEM((RPT, L), jnp.float32),
        pltpu.VMEM((RPT, L), jnp.float32),
        pltpu.VMEM((RPT, L), jnp.float32),
        pltpu.VMEM((L,), jnp.float32),
    ],
    compiler_params=_SC_PARAMS,
)
def _layer2_kernel(s1p_hbm, q1_hbm, dinv_hbm, b1_hbm, edges_hbm,
                   out_hbm,
                   src_v, dst_v, buf, zero_v, q_sh, s_sh,
                   gs0, gs1, gs2, gs3, ss0, ss1, ss2, ss3,
                   s0_v, s1_v, q1_v, dinv_v, q2_v, b1_v):
    cid = lax.axis_index("c")
    sid = lax.axis_index("s")
    wid = cid * NS + sid
    rows = pl.ds(sid * RPT, RPT)
    e0 = pltpu.async_copy(edges_hbm.at[0, wid], src_v, gs0)
    e1 = pltpu.async_copy(edges_hbm.at[1, wid], dst_v, gs1)
    ca = pltpu.async_copy(s1p_hbm.at[0, rows], s0_v, gs2)
    cb = pltpu.async_copy(s1p_hbm.at[1, rows], s1_v, gs3)
    cq = pltpu.async_copy(q1_hbm.at[rows], q1_v, ss0)
    cd = pltpu.async_copy(dinv_hbm.at[rows], dinv_v, ss1)
    cbias = pltpu.async_copy(b1_hbm, b1_v, ss2)
    _fill_zero(zero_v)
    cz = pltpu.async_copy(zero_v, s_sh.at[rows], ss3)
    ca.wait()
    cb.wait()
    cq.wait()
    cd.wait()
    cbias.wait()

    @pl.loop(0, RPT, unroll=8)
    def _(i):
        y = dinv_v[i, :]
        agg = y * (s0_v[i, :] + s1_v[i, :] + q1_v[i, :])
        h = jnp.maximum(agg + b1_v[...], 0.0)
        q2_v[i, :] = y * h

    pltpu.sync_copy(q2_v, q_sh.at[rows])
    e0.wait()
    e1.wait()
    cz.wait()
    plsc.subcore_barrier()
    _agg_pipeline(src_v, dst_v, buf, q_sh, s_sh,
                  (gs0, gs1, gs2, gs3), (ss0, ss1, ss2, ss3))
    plsc.subcore_barrier()

    pltpu.sync_copy(s_sh.at[rows], s0_v)

    @pl.when(cid == 0)
    def _():
        @pl.loop(0, RPT, unroll=8)
        def _(i):
            s0_v[i, :] = dinv_v[i, :] * (s0_v[i, :] + q2_v[i, :])

    @pl.when(cid != 0)
    def _():
        @pl.loop(0, RPT, unroll=8)
        def _(i):
            s0_v[i, :] = dinv_v[i, :] * s0_v[i, :]
    pltpu.sync_copy(s0_v, out_hbm.at[cid, rows])


def _edge_body(e_ref, o_ref):
    o_ref[:, :2500, :] = e_ref[...].reshape(2, 2500, 128)
    lane = jax.lax.broadcasted_iota(jnp.int32, (2, 60, 128), 2)
    row = jax.lax.broadcasted_iota(jnp.int32, (2, 60, 128), 1)
    o_ref[:, 2500:, :] = N + (row * 128 + lane) % (NP - N)


_tc_edges = pl.pallas_call(
    _edge_body,
    in_specs=[pl.BlockSpec((2, E), lambda: (0, 0))],
    out_specs=pl.BlockSpec((2, 2560, 128), lambda: (0, 0, 0)),
    out_shape=jax.ShapeDtypeStruct((2, 2560, 128), jnp.int32),
)


def _mm1_body(x_ref, w_ref, o_ref):
    o_ref[...] = jnp.dot(x_ref[...], w_ref[...],
                         preferred_element_type=jnp.float32)

    @pl.when(pl.program_id(0) == 4)
    def _():
        o_ref[2048 - (NP - N):, :] = jnp.zeros((NP - N, D_HID), jnp.float32)


_tc_mm1 = pl.pallas_call(
    _mm1_body,
    grid=(5,),
    in_specs=[pl.BlockSpec((2048, D_IN), lambda i: (i, 0)),
              pl.BlockSpec((D_IN, D_HID), lambda i: (0, 0))],
    out_specs=pl.BlockSpec((2048, D_HID), lambda i: (i, 0)),
    out_shape=jax.ShapeDtypeStruct((NP, D_HID), jnp.float32),
)


def _mm2_body(p_ref, w_ref, b_ref, g_ref, gt_ref, o_ref):
    p = p_ref[0] + p_ref[1]
    z = jnp.dot(p, w_ref[...], preferred_element_type=jnp.float32)
    z = z + b_ref[...]
    m = jnp.max(z, axis=1, keepdims=True)
    e = jnp.exp(z - m)
    s = jnp.dot(e, g_ref[...], preferred_element_type=jnp.float32)
    logs = jnp.log(s)
    o_ref[...] = (z - m) - jnp.dot(logs, gt_ref[...],
                                   preferred_element_type=jnp.float32)


_PBLK = 256
_tc_mm2 = pl.pallas_call(
    _mm2_body,
    grid=(NP // 8 // _PBLK,),
    in_specs=[pl.BlockSpec((NC, _PBLK, 128), lambda i: (0, i, 0)),
              pl.BlockSpec((128, 8 * CPACK), lambda i: (0, 0)),
              pl.BlockSpec((1, 8 * CPACK), lambda i: (0, 0)),
              pl.BlockSpec((8 * CPACK, 8), lambda i: (0, 0)),
              pl.BlockSpec((8, 8 * CPACK), lambda i: (0, 0))],
    out_specs=pl.BlockSpec((_PBLK, 8 * CPACK), lambda i: (i, 0)),
    out_shape=jax.ShapeDtypeStruct((NP // 8, 8 * CPACK), jnp.float32),
)

_G_NP = np.kron(np.eye(8, dtype=np.float32),
                np.pad(np.ones((N_CLASSES, 1), np.float32),
                       ((0, CPACK - N_CLASSES), (0, 0))))
_GT_NP = _G_NP.T.copy()


def kernel(x, edge_index, W1, b1, W2, b2):
    edges = _tc_edges(edge_index).reshape(2, NW, NCH, CH)

    degp = _deg_kernel(edges)
    p = _tc_mm1(x, W1)
    s1p, q1, dinv = _layer1_kernel(degp, p, edges)
    s2p = _layer2_kernel(s1p, q1, dinv, b1, edges)

    w2big = jnp.kron(jnp.eye(8, dtype=jnp.float32),
                     jnp.pad(W2, ((0, 0), (0, CPACK - N_CLASSES))))
    b2big = jnp.tile(jnp.pad(b2, (0, CPACK - N_CLASSES)), 8)[None, :]
    out_pack = _tc_mm2(s2p.reshape(NC, NP // 8, 128), w2big, b2big,
                       jnp.asarray(_G_NP), jnp.asarray(_GT_NP))
    return out_pack.reshape(NP, CPACK)[:N, :N_CLASSES]


# speedup vs baseline: 1.3558x; 1.1332x over previous
import functools

import jax
import jax.numpy as jnp
import numpy as np
from jax import lax
from jax.experimental import pallas as pl
from jax.experimental.pallas import tpu as pltpu
from jax.experimental.pallas import tpu_sc as plsc

N = 10000
E = 320000
D_IN = 128
D_HID = 16
N_CLASSES = 40

NC, NS, L = 2, 16, 16
NW = NC * NS
CH = 128
NCH = 80
EPT = NCH * CH
EPAD = NW * EPT

NP = 10240
RPT = NP // NS

CPACK = 48
_MESH = plsc.VectorSubcoreMesh(core_axis_name="c", subcore_axis_name="s")
_SC_PARAMS = pltpu.CompilerParams(use_tc_tiling_on_sc=False,
                                  needs_layout_passes=False)


def _fast_rsqrt(d):
    i = plsc.bitcast(d, jnp.int32)
    y = plsc.bitcast(jnp.int32(0x5F3759DF) - (i >> 1), jnp.float32)
    for _ in range(3):
        y = y * (1.5 - 0.5 * d * y * y)
    return y


def _fill_zero(ref):
    @pl.loop(0, RPT, unroll=8)
    def _(i):
        ref[i, :] = jnp.zeros((L,), jnp.float32)


@functools.partial(
    pl.kernel,
    out_type=jax.ShapeDtypeStruct((NC, NP, L), jnp.float32),
    mesh=_MESH,
    scratch_types=[
        pltpu.VMEM((NCH, CH), jnp.int32),
        pltpu.VMEM((CH, L), jnp.float32),
        pltpu.VMEM((RPT, L), jnp.float32),
        pltpu.VMEM_SHARED((NP, L), jnp.float32),
        pltpu.SemaphoreType.DMA,
        pltpu.SemaphoreType.DMA,
        pltpu.SemaphoreType.DMA,
        pltpu.SemaphoreType.DMA,
    ],
    compiler_params=_SC_PARAMS,
)
def _deg_kernel(edges_hbm, out_hbm, dst_v, ones_v, zero_v, acc_sh,
                ss0, ss1, ss2, ss3):
    cid = lax.axis_index("c")
    sid = lax.axis_index("s")
    wid = cid * NS + sid
    ssems = (ss0, ss1, ss2, ss3)
    pltpu.sync_copy(edges_hbm.at[1, wid], dst_v)

    @pl.loop(0, CH, unroll=8)
    def _(i):
        ones_v[i, :] = jnp.ones((L,), jnp.float32)
    _fill_zero(zero_v)
    pltpu.sync_copy(zero_v, acc_sh.at[pl.ds(sid * RPT, RPT)])
    plsc.subcore_barrier()

    @pl.loop(0, NCH, step=4)
    def _(j):
        for k in range(4):
            idx = j + k

            @pl.when(idx >= 4)
            def _():
                pltpu.make_async_copy(ones_v, acc_sh.at[dst_v.at[0]],
                                      ssems[k]).wait()
            pltpu.async_copy(ones_v, acc_sh.at[dst_v.at[idx]], ssems[k],
                             add=True)
    for k in range(4):
        pltpu.make_async_copy(ones_v, acc_sh.at[dst_v.at[0]], ssems[k]).wait()
    plsc.subcore_barrier()
    pltpu.sync_copy(acc_sh.at[pl.ds(sid * RPT, RPT)],
                    out_hbm.at[cid, pl.ds(sid * RPT, RPT)])


def _agg_pipeline(src_v, dst_v, buf, q_sh, s_sh, gsems, ssems):
    def gather(idx, k):
        pltpu.async_copy(q_sh.at[src_v.at[idx]], buf.at[k], gsems[k])

    def wait_gather(k):
        pltpu.make_async_copy(q_sh.at[src_v.at[0]], buf.at[k],
                              gsems[k]).wait()

    def scatter(idx, k):
        pltpu.async_copy(buf.at[k], s_sh.at[dst_v.at[idx]], ssems[k],
                         add=True)

    def wait_scatter(k):
        pltpu.make_async_copy(buf.at[k], s_sh.at[dst_v.at[0]],
                              ssems[k]).wait()

    gather(0, 0)
    gather(1, 1)

    @pl.loop(0, NCH, step=4)
    def _(j):
        for k in range(4):
            idx = j + k
            wait_gather(k)
            scatter(idx, k)
            k2 = (k + 2) % 4

            @pl.when(idx + 2 < NCH)
            def _():
                @pl.when(idx >= 2)
                def _():
                    wait_scatter(k2)
                gather(idx + 2, k2)
    for k in range(4):
        wait_scatter(k)


_AGG_SCRATCH = [
    pltpu.VMEM((NCH, CH), jnp.int32),
    pltpu.VMEM((NCH, CH), jnp.int32),
    pltpu.VMEM((4, CH, L), jnp.float32),
    pltpu.VMEM((RPT, L), jnp.float32),
    pltpu.VMEM_SHARED((NP, L), jnp.float32),
    pltpu.VMEM_SHARED((NP, L), jnp.float32),
    pltpu.SemaphoreType.DMA,
    pltpu.SemaphoreType.DMA,
    pltpu.SemaphoreType.DMA,
    pltpu.SemaphoreType.DMA,
    pltpu.SemaphoreType.DMA,
    pltpu.SemaphoreType.DMA,
    pltpu.SemaphoreType.DMA,
    pltpu.SemaphoreType.DMA,
]


@functools.partial(
    pl.kernel,
    out_type=(jax.ShapeDtypeStruct((NC, NP, L), jnp.float32),
              jax.ShapeDtypeStruct((NP, L), jnp.float32),
              jax.ShapeDtypeStruct((NP, L), jnp.float32)),
    mesh=_MESH,
    scratch_types=_AGG_SCRATCH + [
        pltpu.VMEM((RPT // 8, 8, L), jnp.float32),
        pltpu.VMEM((RPT, L), jnp.float32),
        pltpu.VMEM((RPT, L), jnp.float32),
        pltpu.VMEM((RPT, L), jnp.float32),
    ],
    compiler_params=_SC_PARAMS,
)
def _layer1_kernel(degp_hbm, p_hbm, edges_hbm,
                   out_hbm, q1_hbm, dinv_hbm,
                   src_v, dst_v, buf, zero_v, q_sh, s_sh,
                   gs0, gs1, gs2, gs3, ss0, ss1, ss2, ss3,
                   p_v, d0_v, d1_v, q_v):
    cid = lax.axis_index("c")
    sid = lax.axis_index("s")
    wid = cid * NS + sid
    rows = pl.ds(sid * RPT, RPT)
    e0 = pltpu.async_copy(edges_hbm.at[0, wid], src_v, gs0)
    e1 = pltpu.async_copy(edges_hbm.at[1, wid], dst_v, gs1)
    prows = pl.ds(sid * (RPT // 8), RPT // 8)
    cp = pltpu.async_copy(p_hbm.at[prows, :, pl.ds(0, L)], p_v, gs2)
    c0 = pltpu.async_copy(degp_hbm.at[0, rows], d0_v, gs3)
    c1 = pltpu.async_copy(degp_hbm.at[1, rows], d1_v, ss0)
    _fill_zero(zero_v)
    cz = pltpu.async_copy(zero_v, s_sh.at[rows], ss1)
    cp.wait()
    c0.wait()
    c1.wait()

    @pl.loop(0, RPT // 8)
    def _(g):
        for a in range(8):
            i = g * 8 + a
            d = d0_v[i, :] + d1_v[i, :] + 1.0
            y = _fast_rsqrt(d)
            d1_v[i, :] = y
            q_v[i, :] = y * p_v[g, a, :]

    pltpu.sync_copy(q_v, q_sh.at[rows])

    @pl.when(cid == 0)
    def _():
        pltpu.sync_copy(q_v, q1_hbm.at[rows])
        pltpu.sync_copy(d1_v, dinv_hbm.at[rows])

    e0.wait()
    e1.wait()
    cz.wait()
    plsc.subcore_barrier()
    _agg_pipeline(src_v, dst_v, buf, q_sh, s_sh,
                  (gs0, gs1, gs2, gs3), (ss0, ss1, ss2, ss3))
    plsc.subcore_barrier()
    pltpu.sync_copy(s_sh.at[rows], out_hbm.at[cid, rows])


@functools.partial(
    pl.kernel,
    out_type=jax.ShapeDtypeStruct((NC, NP, L), jnp.float32),
    mesh=_MESH,
    scratch_types=_AGG_SCRATCH + [
        pltpu.VMEM((RPT, L), jnp.float32),
        pltpu.VMEM((RPT, L), jnp.float32),
        pltpu.VMEM((RPT, L), jnp.float32),
        pltpu.VMEM((RPT, L), jnp.float32),
        pltpu.VMEM((RPT, L), jnp.float32),
        pltpu.VMEM((L,), jnp.float32),
    ],
    compiler_params=_SC_PARAMS,
)
def _layer2_kernel(s1p_hbm, q1_hbm, dinv_hbm, b1_hbm, edges_hbm,
                   out_hbm,
                   src_v, dst_v, buf, zero_v, q_sh, s_sh,
                   gs0, gs1, gs2, gs3, ss0, ss1, ss2, ss3,
                   s0_v, s1_v, q1_v, dinv_v, q2_v, b1_v):
    cid = lax.axis_index("c")
    sid = lax.axis_index("s")
    wid = cid * NS + sid
    rows = pl.ds(sid * RPT, RPT)
    e0 = pltpu.async_copy(edges_hbm.at[0, wid], src_v, gs0)
    e1 = pltpu.async_copy(edges_hbm.at[1, wid], dst_v, gs1)
    ca = pltpu.async_copy(s1p_hbm.at[0, rows], s0_v, gs2)
    cb = pltpu.async_copy(s1p_hbm.at[1, rows], s1_v, gs3)
    cq = pltpu.async_copy(q1_hbm.at[rows], q1_v, ss0)
    cd = pltpu.async_copy(dinv_hbm.at[rows], dinv_v, ss1)
    cbias = pltpu.async_copy(b1_hbm, b1_v, ss2)
    _fill_zero(zero_v)
    cz = pltpu.async_copy(zero_v, s_sh.at[rows], ss3)
    ca.wait()
    cb.wait()
    cq.wait()
    cd.wait()
    cbias.wait()

    @pl.loop(0, RPT, unroll=8)
    def _(i):
        y = dinv_v[i, :]
        agg = y * (s0_v[i, :] + s1_v[i, :] + q1_v[i, :])
        h = jnp.maximum(agg + b1_v[...], 0.0)
        q2_v[i, :] = y * h

    pltpu.sync_copy(q2_v, q_sh.at[rows])
    e0.wait()
    e1.wait()
    cz.wait()
    plsc.subcore_barrier()
    _agg_pipeline(src_v, dst_v, buf, q_sh, s_sh,
                  (gs0, gs1, gs2, gs3), (ss0, ss1, ss2, ss3))
    plsc.subcore_barrier()

    pltpu.sync_copy(s_sh.at[rows], s0_v)

    @pl.when(cid == 0)
    def _():
        @pl.loop(0, RPT, unroll=8)
        def _(i):
            s0_v[i, :] = dinv_v[i, :] * (s0_v[i, :] + q2_v[i, :])

    @pl.when(cid != 0)
    def _():
        @pl.loop(0, RPT, unroll=8)
        def _(i):
            s0_v[i, :] = dinv_v[i, :] * s0_v[i, :]
    pltpu.sync_copy(s0_v, out_hbm.at[cid, rows])


def _edge_body(e_ref, o_ref):
    o_ref[:, :2500, :] = e_ref[...].reshape(2, 2500, 128)
    lane = jax.lax.broadcasted_iota(jnp.int32, (2, 60, 128), 2)
    row = jax.lax.broadcasted_iota(jnp.int32, (2, 60, 128), 1)
    o_ref[:, 2500:, :] = N + (row * 128 + lane) % (NP - N)


_tc_edges = pl.pallas_call(
    _edge_body,
    in_specs=[pl.BlockSpec((2, E), lambda: (0, 0))],
    out_specs=pl.BlockSpec((2, 2560, 128), lambda: (0, 0, 0)),
    out_shape=jax.ShapeDtypeStruct((2, 2560, 128), jnp.int32),
)


def _mm1_body(x_ref, w_ref, o_ref):
    p = jnp.dot(x_ref[...], w_ref[...], preferred_element_type=jnp.float32)
    r = jax.lax.broadcasted_iota(jnp.int32, (2048, D_HID), 0)
    p = jnp.where((pl.program_id(0) < 4) | (r < 2048 - (NP - N)), p, 0.0)
    o_ref[:, :, 0:D_HID] = p.reshape(256, 8, D_HID)


_tc_mm1 = pl.pallas_call(
    _mm1_body,
    grid=(5,),
    in_specs=[pl.BlockSpec((2048, D_IN), lambda i: (i, 0)),
              pl.BlockSpec((D_IN, D_HID), lambda i: (0, 0))],
    out_specs=pl.BlockSpec((256, 8, 128), lambda i: (i, 0, 0)),
    out_shape=jax.ShapeDtypeStruct((NP // 8, 8, 128), jnp.float32),
)


def _mm2_body(p_ref, w_ref, b_ref, g_ref, gt_ref, o_ref):
    p = p_ref[0] + p_ref[1]
    z = jnp.dot(p, w_ref[...], preferred_element_type=jnp.float32)
    z = z + b_ref[...]
    m = jnp.max(z, axis=1, keepdims=True)
    e = jnp.exp(z - m)
    s = jnp.dot(e, g_ref[...], preferred_element_type=jnp.float32)
    logs = jnp.log(s)
    o_ref[...] = (z - m) - jnp.dot(logs, gt_ref[...],
                                   preferred_element_type=jnp.float32)


_PBLK = 256
_tc_mm2 = pl.pallas_call(
    _mm2_body,
    grid=(NP // 8 // _PBLK,),
    in_specs=[pl.BlockSpec((NC, _PBLK, 128), lambda i: (0, i, 0)),
              pl.BlockSpec((128, 8 * CPACK), lambda i: (0, 0)),
              pl.BlockSpec((1, 8 * CPACK), lambda i: (0, 0)),
              pl.BlockSpec((8 * CPACK, 8), lambda i: (0, 0)),
              pl.BlockSpec((8, 8 * CPACK), lambda i: (0, 0))],
    out_specs=pl.BlockSpec((_PBLK, 8 * CPACK), lambda i: (i, 0)),
    out_shape=jax.ShapeDtypeStruct((NP // 8, 8 * CPACK), jnp.float32),
)

_G_NP = np.kron(np.eye(8, dtype=np.float32),
                np.pad(np.ones((N_CLASSES, 1), np.float32),
                       ((0, CPACK - N_CLASSES), (0, 0))))
_GT_NP = _G_NP.T.copy()


def kernel(x, edge_index, W1, b1, W2, b2):
    edges = _tc_edges(edge_index).reshape(2, NW, NCH, CH)

    degp = _deg_kernel(edges)
    p = _tc_mm1(x, W1)
    s1p, q1, dinv = _layer1_kernel(degp, p, edges)
    s2p = _layer2_kernel(s1p, q1, dinv, b1, edges)

    w2big = jnp.kron(jnp.eye(8, dtype=jnp.float32),
                     jnp.pad(W2, ((0, 0), (0, CPACK - N_CLASSES))))
    b2big = jnp.tile(jnp.pad(b2, (0, CPACK - N_CLASSES)), 8)[None, :]
    out_pack = _tc_mm2(s2p.reshape(NC, NP // 8, 128), w2big, b2big,
                       jnp.asarray(_G_NP), jnp.asarray(_GT_NP))
    return out_pack.reshape(NP, CPACK)[:N, :N_CLASSES]
